# Initial kernel scaffold; baseline (speedup 1.0000x reference)
import functools

import jax
import jax.numpy as jnp
from jax import lax
from jax.experimental import pallas as pl
from jax.experimental.pallas import tpu as pltpu
from jax.experimental.pallas import tpu_sc as plsc

N = 10000
D = 128
E = 320000

NC = 2
NS = 16
NW = NC * NS
B = 128
J = 4
EPT = 10240
ROWS_PT = EPT // B
N_PAD = 10016
RPT = N_PAD // NS

_mesh = plsc.VectorSubcoreMesh(core_axis_name="c", subcore_axis_name="s")


@functools.partial(
    pl.kernel,
    out_type=jax.ShapeDtypeStruct((NC, N_PAD, D), jnp.float32),
    mesh=_mesh,
    scratch_types=[
        pltpu.VMEM((J, B), jnp.int32),
        pltpu.VMEM((J, B), jnp.int32),
        pltpu.VMEM((J * B, D), jnp.float32),
        pltpu.VMEM_SHARED((N_PAD, D), jnp.float32),
        pltpu.SemaphoreType.DMA,
    ],
)
def _sc_segsum(table, src_r, dst_r, zeros, out, src_v, dst_v, rows_v, acc, sem):
    c = lax.axis_index("c")
    s = lax.axis_index("s")
    w = c * NS + s

    pltpu.sync_copy(zeros.at[pl.ds(s * RPT, RPT)], acc.at[pl.ds(s * RPT, RPT)])
    plsc.subcore_barrier()

    def chunk(ch, carry):
        pltpu.sync_copy(src_r.at[w, pl.ds(ch * J, J)], src_v)
        pltpu.sync_copy(dst_r.at[w, pl.ds(ch * J, J)], dst_v)
        gathers = [
            pltpu.async_copy(table.at[src_v.at[j]],
                             rows_v.at[pl.ds(j * B, B)], sem)
            for j in range(J)
        ]
        for j in range(J):
            gathers[j].wait()
            pltpu.sync_copy(rows_v.at[pl.ds(j * B, B)], acc.at[dst_v.at[j]],
                            add=True)
        return carry

    lax.fori_loop(0, ROWS_PT // J, chunk, 0)

    plsc.subcore_barrier()
    pltpu.sync_copy(acc.at[pl.ds(s * RPT, RPT)], out.at[c, pl.ds(s * RPT, RPT)])


@functools.partial(
    pl.kernel,
    out_type=jax.ShapeDtypeStruct((NC, N_PAD, 16), jnp.float32),
    mesh=_mesh,
    scratch_types=[
        pltpu.VMEM((J, B), jnp.int32),
        pltpu.VMEM((B, 16), jnp.float32),
        pltpu.VMEM_SHARED((N_PAD, 16), jnp.float32),
    ],
)
def _sc_counts(dst_r, zeros16, ones16, out, dst_v, ones_v, acc):
    c = lax.axis_index("c")
    s = lax.axis_index("s")
    w = c * NS + s

    pltpu.sync_copy(zeros16.at[pl.ds(s * RPT, RPT)], acc.at[pl.ds(s * RPT, RPT)])
    pltpu.sync_copy(ones16, ones_v)
    plsc.subcore_barrier()

    def chunk(ch, carry):
        pltpu.sync_copy(dst_r.at[w, pl.ds(ch * J, J)], dst_v)
        for j in range(J):
            pltpu.sync_copy(ones_v, acc.at[dst_v.at[j]], add=True)
        return carry

    lax.fori_loop(0, ROWS_PT // J, chunk, 0)

    plsc.subcore_barrier()
    pltpu.sync_copy(acc.at[pl.ds(s * RPT, RPT)], out.at[c, pl.ds(s * RPT, RPT)])


def _matmul_t_body(x_ref, w_ref, o_ref):
    o_ref[:] = lax.dot_general(x_ref[:], w_ref[:], (((1,), (1,)), ((), ())),
                               preferred_element_type=jnp.float32)


def _tc_matmul_t(x, w):
    return pl.pallas_call(
        _matmul_t_body,
        out_shape=jax.ShapeDtypeStruct((x.shape[0], w.shape[0]), jnp.float32),
    )(x, w)


def _combine_body(relu, seg_ref, cnt_ref, x_ref, wr_ref, bl_ref, o_ref):
    cnt = jnp.sum(cnt_ref[0] + cnt_ref[1], axis=1, keepdims=True) * (1.0 / 16.0)
    mean = (seg_ref[0] + seg_ref[1]) / jnp.maximum(cnt, 1.0)
    root = lax.dot_general(x_ref[:], wr_ref[:], (((1,), (1,)), ((), ())),
                           preferred_element_type=jnp.float32)
    o = mean + bl_ref[:] + root
    if relu:
        o = jnp.maximum(o, 0.0)
    o_ref[:] = o


def _tc_combine(seg, cnt, x, wr, bl, relu):
    return pl.pallas_call(
        functools.partial(_combine_body, relu),
        out_shape=jax.ShapeDtypeStruct((N, D), jnp.float32),
    )(seg, cnt, x, wr, bl)


def kernel(x, edge_index, W1l, b1l, W1r, W2l, b2l, W2r):
    src = edge_index[0]
    dst = edge_index[1]
    pad = NW * EPT - E
    src_r = jnp.concatenate([src, jnp.zeros((pad,), jnp.int32)]).reshape(NW, ROWS_PT, B)
    dst_r = jnp.concatenate([dst, jnp.full((pad,), N, jnp.int32)]).reshape(NW, ROWS_PT, B)

    zeros = jnp.zeros((N_PAD, D), jnp.float32)
    zeros16 = jnp.zeros((N_PAD, 16), jnp.float32)
    ones16 = jnp.ones((B, 16), jnp.float32)

    cnt_raw = _sc_counts(dst_r, zeros16, ones16)
    cnt = cnt_raw[:, :N]

    t1 = _tc_matmul_t(x, W1l)
    seg1 = _sc_segsum(t1, src_r, dst_r, zeros)[:, :N]
    h = _tc_combine(seg1, cnt, x, W1r, b1l.reshape(1, D), relu=True)

    t2 = _tc_matmul_t(h, W2l)
    seg2 = _sc_segsum(t2, src_r, dst_r, zeros)[:, :N]
    out = _tc_combine(seg2, cnt, h, W2r, b2l.reshape(1, D), relu=False)
    return out


# speedup vs baseline: 3.0208x; 3.0208x over previous
import functools

import jax
import jax.numpy as jnp
from jax import lax
from jax.experimental import pallas as pl
from jax.experimental.pallas import tpu as pltpu
from jax.experimental.pallas import tpu_sc as plsc

N = 10000
D = 128
E = 320000

NC = 2
NS = 16
NW = NC * NS
B = 128
L = 16
EPT = 10240
ROWS_PT = EPT // B
N_PAD = 10112
RPT = N_PAD // NS

_mesh = plsc.VectorSubcoreMesh(core_axis_name="c", subcore_axis_name="s")


def _make_segsum(with_counts):
    out_type = [jax.ShapeDtypeStruct((NC, N_PAD, D), jnp.float32)]
    scratch = [
        pltpu.VMEM((B,), jnp.int32),
        pltpu.VMEM((B,), jnp.int32),
        pltpu.VMEM((B, D), jnp.float32),
        pltpu.VMEM_SHARED((N_PAD, D), jnp.float32),
        pltpu.SemaphoreType.DMA,
    ]
    if with_counts:
        out_type.append(jax.ShapeDtypeStruct((NW, N_PAD), jnp.float32))
        scratch.append(pltpu.VMEM((N_PAD,), jnp.float32))

    @functools.partial(pl.kernel, out_type=out_type, mesh=_mesh,
                       scratch_types=scratch,
                       compiler_params=pltpu.CompilerParams(
                           needs_layout_passes=False))
    def segsum(table, src_r, dst_r, zeros, out, *rest):
        if with_counts:
            cnt_out, src_v, dst_v, rows_v, acc, sem, cnt_v = rest
        else:
            src_v, dst_v, rows_v, acc, sem = rest
        c = lax.axis_index("c")
        s = lax.axis_index("s")
        w = c * NS + s

        pltpu.sync_copy(zeros.at[pl.ds(s * RPT, RPT)], acc.at[pl.ds(s * RPT, RPT)])
        if with_counts:
            def zc(i, carry):
                cnt_v[pl.ds(i * L, L)] = jnp.zeros((L,), jnp.float32)
                return carry
            lax.fori_loop(0, N_PAD // L, zc, 0)
        plsc.subcore_barrier()

        ones_l = jnp.full((L,), 1.0, jnp.float32)

        def chunk(ch, carry):
            pltpu.sync_copy(src_r.at[w, ch], src_v)
            pltpu.sync_copy(dst_r.at[w, ch], dst_v)
            pltpu.async_copy(table.at[src_v], rows_v, sem).wait()
            pltpu.sync_copy(rows_v, acc.at[dst_v], add=True)
            if with_counts:
                for k in range(B // L):
                    plsc.addupdate_scatter(
                        cnt_v, [dst_v[pl.ds(k * L, L)]], ones_l)
            return carry

        lax.fori_loop(0, ROWS_PT, chunk, 0)

        plsc.subcore_barrier()
        pltpu.sync_copy(acc.at[pl.ds(s * RPT, RPT)], out.at[c, pl.ds(s * RPT, RPT)])
        if with_counts:
            pltpu.sync_copy(cnt_v, cnt_out.at[w])

    return segsum


_sc_segsum_cnt = _make_segsum(True)
_sc_segsum = _make_segsum(False)


def _matmul_t_body(x_ref, w_ref, o_ref):
    o_ref[:] = lax.dot_general(x_ref[:], w_ref[:], (((1,), (1,)), ((), ())),
                               preferred_element_type=jnp.float32)


def _tc_matmul_t(x, w):
    return pl.pallas_call(
        _matmul_t_body,
        out_shape=jax.ShapeDtypeStruct((x.shape[0], w.shape[0]), jnp.float32),
    )(x, w)


def _combine_body(relu, seg_ref, cnt_ref, x_ref, wr_ref, bl_ref, o_ref):
    cnt = jnp.reshape(jnp.sum(cnt_ref[:], axis=0), (N, 1))
    mean = (seg_ref[0] + seg_ref[1]) / jnp.maximum(cnt, 1.0)
    root = lax.dot_general(x_ref[:], wr_ref[:], (((1,), (1,)), ((), ())),
                           preferred_element_type=jnp.float32)
    o = mean + bl_ref[:] + root
    if relu:
        o = jnp.maximum(o, 0.0)
    o_ref[:] = o


def _tc_combine(seg, cnt, x, wr, bl, relu):
    return pl.pallas_call(
        functools.partial(_combine_body, relu),
        out_shape=jax.ShapeDtypeStruct((N, D), jnp.float32),
    )(seg, cnt, x, wr, bl)


def kernel(x, edge_index, W1l, b1l, W1r, W2l, b2l, W2r):
    src = edge_index[0]
    dst = edge_index[1]
    pad = NW * EPT - E
    src_r = jnp.concatenate([src, jnp.zeros((pad,), jnp.int32)]).reshape(NW, ROWS_PT, B)
    dst_r = jnp.concatenate([dst, jnp.full((pad,), N, jnp.int32)]).reshape(NW, ROWS_PT, B)

    zeros = jnp.zeros((N_PAD, D), jnp.float32)

    t1 = _tc_matmul_t(x, W1l)
    seg1, cnt_raw = _sc_segsum_cnt(t1, src_r, dst_r, zeros)
    cnt = cnt_raw[:, :N]
    h = _tc_combine(seg1[:, :N], cnt, x, W1r, b1l.reshape(1, D), relu=True)

    t2 = _tc_matmul_t(h, W2l)
    (seg2,) = _sc_segsum(t2, src_r, dst_r, zeros)
    out = _tc_combine(seg2[:, :N], cnt, h, W2r, b2l.reshape(1, D), relu=False)
    return out


# speedup vs baseline: 3.5186x; 1.1648x over previous
import functools

import jax
import jax.numpy as jnp
from jax import lax
from jax.experimental import pallas as pl
from jax.experimental.pallas import tpu as pltpu
from jax.experimental.pallas import tpu_sc as plsc

N = 10000
D = 128
E = 320000

NC = 2
NS = 16
NW = NC * NS
B = 128
L = 16
EPT = 10240
ROWS_PT = EPT // B
N_PAD = 10112
RPT = N_PAD // NS

_mesh = plsc.VectorSubcoreMesh(core_axis_name="c", subcore_axis_name="s")


def _make_segsum(with_counts):
    out_type = [jax.ShapeDtypeStruct((NC, N_PAD, D), jnp.float32)]
    scratch = [
        pltpu.VMEM((2, B), jnp.int32),
        pltpu.VMEM((2, B), jnp.int32),
        pltpu.VMEM((B, D), jnp.float32),
        pltpu.VMEM((B, D), jnp.float32),
        pltpu.VMEM_SHARED((N_PAD, D), jnp.float32),
        pltpu.SemaphoreType.DMA,
        pltpu.SemaphoreType.DMA,
        pltpu.SemaphoreType.DMA,
        pltpu.SemaphoreType.DMA,
    ]
    if with_counts:
        out_type.append(jax.ShapeDtypeStruct((NW, N_PAD), jnp.float32))
        scratch.append(pltpu.VMEM((N_PAD,), jnp.float32))

    @functools.partial(pl.kernel, out_type=out_type, mesh=_mesh,
                       scratch_types=scratch,
                       compiler_params=pltpu.CompilerParams(
                           needs_layout_passes=False))
    def segsum(table, edge_r, zeros, out, *rest):
        if with_counts:
            cnt_out, ixa, ixb, rva, rvb, acc, gsa, gsb, ssa, ssb, cnt_v = rest
        else:
            ixa, ixb, rva, rvb, acc, gsa, gsb, ssa, ssb = rest
        c = lax.axis_index("c")
        s = lax.axis_index("s")
        w = c * NS + s

        pltpu.sync_copy(zeros.at[pl.ds(s * RPT, RPT)], acc.at[pl.ds(s * RPT, RPT)])
        if with_counts:
            def zc(i, carry):
                cnt_v[pl.ds(i * L, L)] = jnp.zeros((L,), jnp.float32)
                return carry
            lax.fori_loop(0, N_PAD // L, zc, 0)
        plsc.subcore_barrier()

        ones_l = jnp.full((L,), 1.0, jnp.float32)

        def count(ix):
            if with_counts:
                for k in range(B // L):
                    plsc.addupdate_scatter(
                        cnt_v, [ix[1, pl.ds(k * L, L)]], ones_l)

        pltpu.sync_copy(edge_r.at[w, 0], ixa)
        pltpu.async_copy(table.at[ixa.at[0]], rva, gsa)

        def pair(i, carry):
            pltpu.sync_copy(edge_r.at[w, 2 * i + 1], ixb)
            pltpu.async_copy(table.at[ixb.at[0]], rvb, gsb)
            pltpu.make_async_copy(table.at[ixa.at[0]], rva, gsa).wait()
            pltpu.async_copy(rva, acc.at[ixa.at[1]], ssa, add=True)
            count(ixa)
            pltpu.make_async_copy(table.at[ixb.at[0]], rvb, gsb).wait()
            pltpu.async_copy(rvb, acc.at[ixb.at[1]], ssb, add=True)
            count(ixb)
            pltpu.make_async_copy(rva, acc.at[ixa.at[1]], ssa).wait()

            @pl.when(i + 1 < ROWS_PT // 2)
            def _():
                pltpu.sync_copy(edge_r.at[w, 2 * i + 2], ixa)
                pltpu.async_copy(table.at[ixa.at[0]], rva, gsa)

            pltpu.make_async_copy(rvb, acc.at[ixb.at[1]], ssb).wait()
            return carry

        lax.fori_loop(0, ROWS_PT // 2, pair, 0)

        plsc.subcore_barrier()
        pltpu.sync_copy(acc.at[pl.ds(s * RPT, RPT)], out.at[c, pl.ds(s * RPT, RPT)])
        if with_counts:
            pltpu.sync_copy(cnt_v, cnt_out.at[w])

    return segsum


_sc_segsum_cnt = _make_segsum(True)
_sc_segsum = _make_segsum(False)


def _matmul_t_body(x_ref, w_ref, o_ref):
    o_ref[:] = lax.dot_general(x_ref[:], w_ref[:], (((1,), (1,)), ((), ())),
                               preferred_element_type=jnp.float32)


def _tc_matmul_t(x, w):
    return pl.pallas_call(
        _matmul_t_body,
        out_shape=jax.ShapeDtypeStruct((x.shape[0], w.shape[0]), jnp.float32),
    )(x, w)


def _combine_body(relu, seg_ref, cnt_ref, x_ref, wr_ref, bl_ref, o_ref):
    cnt = jnp.reshape(jnp.sum(cnt_ref[:], axis=0), (N, 1))
    mean = (seg_ref[0] + seg_ref[1]) / jnp.maximum(cnt, 1.0)
    root = lax.dot_general(x_ref[:], wr_ref[:], (((1,), (1,)), ((), ())),
                           preferred_element_type=jnp.float32)
    o = mean + bl_ref[:] + root
    if relu:
        o = jnp.maximum(o, 0.0)
    o_ref[:] = o


def _tc_combine(seg, cnt, x, wr, bl, relu):
    return pl.pallas_call(
        functools.partial(_combine_body, relu),
        out_shape=jax.ShapeDtypeStruct((N, D), jnp.float32),
    )(seg, cnt, x, wr, bl)


def kernel(x, edge_index, W1l, b1l, W1r, W2l, b2l, W2r):
    src = edge_index[0]
    dst = edge_index[1]
    pad = NW * EPT - E
    src_r = jnp.concatenate([src, jnp.zeros((pad,), jnp.int32)]).reshape(NW, ROWS_PT, B)
    dst_r = jnp.concatenate([dst, jnp.full((pad,), N, jnp.int32)]).reshape(NW, ROWS_PT, B)
    edge_r = jnp.stack([src_r, dst_r], axis=2)

    zeros = jnp.zeros((N_PAD, D), jnp.float32)

    t1 = _tc_matmul_t(x, W1l)
    seg1, cnt_raw = _sc_segsum_cnt(t1, edge_r, zeros)
    cnt = cnt_raw[:, :N]
    h = _tc_combine(seg1[:, :N], cnt, x, W1r, b1l.reshape(1, D), relu=True)

    t2 = _tc_matmul_t(h, W2l)
    (seg2,) = _sc_segsum(t2, edge_r, zeros)
    out = _tc_combine(seg2[:, :N], cnt, h, W2r, b2l.reshape(1, D), relu=False)
    return out


# speedup vs baseline: 4.0528x; 1.1518x over previous
import functools

import jax
import jax.numpy as jnp
from jax import lax
from jax.experimental import pallas as pl
from jax.experimental.pallas import tpu as pltpu
from jax.experimental.pallas import tpu_sc as plsc

N = 10000
D = 128
E = 320000

NC = 2
NS = 16
NW = NC * NS
B = 64
NB = 4
L = 16
EPT = 10240
ROWS_PT = EPT // B
N_PAD = 10112
RPT = N_PAD // NS

_mesh = plsc.VectorSubcoreMesh(core_axis_name="c", subcore_axis_name="s")


def _make_segsum(with_counts):
    out_type = [jax.ShapeDtypeStruct((NC, N_PAD, D), jnp.float32)]
    scratch = (
        [pltpu.VMEM((2, B), jnp.int32) for _ in range(NB)]
        + [pltpu.VMEM((B, D), jnp.float32) for _ in range(NB)]
        + [pltpu.VMEM_SHARED((N_PAD, D), jnp.float32)]
        + [pltpu.SemaphoreType.DMA for _ in range(NB)]
        + [pltpu.SemaphoreType.DMA for _ in range(NB)]
    )
    if with_counts:
        out_type.append(jax.ShapeDtypeStruct((NW, N_PAD), jnp.float32))
        scratch.append(pltpu.VMEM((N_PAD,), jnp.float32))

    @functools.partial(pl.kernel, out_type=out_type, mesh=_mesh,
                       scratch_types=scratch,
                       compiler_params=pltpu.CompilerParams(
                           needs_layout_passes=False))
    def segsum(table, edge_r, zeros, out, *rest):
        if with_counts:
            cnt_out = rest[0]
            rest = rest[1:]
        ix = rest[0:NB]
        rv = rest[NB:2 * NB]
        acc = rest[2 * NB]
        gs = rest[2 * NB + 1:3 * NB + 1]
        ss = rest[3 * NB + 1:4 * NB + 1]
        cnt_v = rest[4 * NB + 1] if with_counts else None
        c = lax.axis_index("c")
        s = lax.axis_index("s")
        w = c * NS + s

        pltpu.sync_copy(zeros.at[pl.ds(s * RPT, RPT)], acc.at[pl.ds(s * RPT, RPT)])
        if with_counts:
            def zc(i, carry):
                cnt_v[pl.ds(i * L, L)] = jnp.zeros((L,), jnp.float32)
                return carry
            lax.fori_loop(0, N_PAD // L, zc, 0)
        plsc.subcore_barrier()

        ones_l = jnp.full((L,), 1.0, jnp.float32)

        def count(ixj):
            if with_counts:
                for k in range(B // L):
                    plsc.addupdate_scatter(
                        cnt_v, [ixj[1, pl.ds(k * L, L)]], ones_l)

        def start_gather(j, ch):
            pltpu.sync_copy(edge_r.at[w, ch], ix[j])
            pltpu.async_copy(table.at[ix[j].at[0]], rv[j], gs[j])

        def wait_gather(j):
            pltpu.make_async_copy(table.at[ix[j].at[0]], rv[j], gs[j]).wait()

        def start_scatter(j):
            pltpu.async_copy(rv[j], acc.at[ix[j].at[1]], ss[j], add=True)

        def wait_scatter(j):
            pltpu.make_async_copy(rv[j], acc.at[ix[j].at[1]], ss[j]).wait()

        start_gather(0, 0)
        start_gather(1, 1)

        def slot(i, j, _):
            ch = NB * i + j
            wait_gather(j)
            start_scatter(j)
            count(ix[j])
            jq = (j + 2) % NB
            @pl.when(ch >= 2)
            def _():
                wait_scatter(jq)
            @pl.when(ch + 2 < ROWS_PT)
            def _():
                start_gather(jq, ch + 2)
            return _

        def ring(i, carry):
            for j in range(NB):
                slot(i, j, carry)
            return carry

        lax.fori_loop(0, ROWS_PT // NB, ring, 0)

        wait_scatter((ROWS_PT - 2) % NB)
        wait_scatter((ROWS_PT - 1) % NB)

        plsc.subcore_barrier()
        pltpu.sync_copy(acc.at[pl.ds(s * RPT, RPT)], out.at[c, pl.ds(s * RPT, RPT)])
        if with_counts:
            pltpu.sync_copy(cnt_v, cnt_out.at[w])

    return segsum


_sc_segsum_cnt = _make_segsum(True)
_sc_segsum = _make_segsum(False)


def _matmul_t_body(x_ref, w_ref, o_ref):
    o_ref[:] = lax.dot_general(x_ref[:], w_ref[:], (((1,), (1,)), ((), ())),
                               preferred_element_type=jnp.float32)


def _tc_matmul_t(x, w):
    return pl.pallas_call(
        _matmul_t_body,
        out_shape=jax.ShapeDtypeStruct((x.shape[0], w.shape[0]), jnp.float32),
    )(x, w)


def _combine_body(relu, seg_ref, cnt_ref, x_ref, wr_ref, bl_ref, o_ref):
    cnt = jnp.reshape(jnp.sum(cnt_ref[:], axis=0), (N, 1))
    mean = (seg_ref[0] + seg_ref[1]) / jnp.maximum(cnt, 1.0)
    root = lax.dot_general(x_ref[:], wr_ref[:], (((1,), (1,)), ((), ())),
                           preferred_element_type=jnp.float32)
    o = mean + bl_ref[:] + root
    if relu:
        o = jnp.maximum(o, 0.0)
    o_ref[:] = o


def _tc_combine(seg, cnt, x, wr, bl, relu):
    return pl.pallas_call(
        functools.partial(_combine_body, relu),
        out_shape=jax.ShapeDtypeStruct((N, D), jnp.float32),
    )(seg, cnt, x, wr, bl)


def kernel(x, edge_index, W1l, b1l, W1r, W2l, b2l, W2r):
    src = edge_index[0]
    dst = edge_index[1]
    pad = NW * EPT - E
    src_r = jnp.concatenate([src, jnp.zeros((pad,), jnp.int32)]).reshape(NW, ROWS_PT, B)
    dst_r = jnp.concatenate([dst, jnp.full((pad,), N, jnp.int32)]).reshape(NW, ROWS_PT, B)
    edge_r = jnp.stack([src_r, dst_r], axis=2)

    zeros = jnp.zeros((N_PAD, D), jnp.float32)

    t1 = _tc_matmul_t(x, W1l)
    seg1, cnt_raw = _sc_segsum_cnt(t1, edge_r, zeros)
    cnt = cnt_raw[:, :N]
    h = _tc_combine(seg1[:, :N], cnt, x, W1r, b1l.reshape(1, D), relu=True)

    t2 = _tc_matmul_t(h, W2l)
    (seg2,) = _sc_segsum(t2, edge_r, zeros)
    out = _tc_combine(seg2[:, :N], cnt, h, W2r, b2l.reshape(1, D), relu=False)
    return out
